# baseline (speedup 1.0000x reference)
import functools

import jax
import jax.numpy as jnp
from jax import lax
from jax.experimental import pallas as pl
from jax.experimental.pallas import tpu as pltpu
from jax.experimental.pallas import tpu_sc as plsc

NC = 2
NS = 16
NW = NC * NS
EB = 128


def _sc_mesh():
    return plsc.VectorSubcoreMesh(
        core_axis_name="c", subcore_axis_name="s", num_cores=NC, num_subcores=NS
    )


def _degree_body(dst_hbm, ones_hbm, zeros_hbm, out_hbm, idx_v, ones_v, acc_sh, sem):
    c = lax.axis_index("c")
    s = lax.axis_index("s")
    wid = c * NS + s
    k = idx_v.shape[0]
    stripe = acc_sh.shape[0] // NS

    pltpu.sync_copy(dst_hbm.at[wid], idx_v)
    pltpu.sync_copy(ones_hbm, ones_v)
    pltpu.sync_copy(zeros_hbm, acc_sh.at[pl.ds(s * stripe, stripe)])
    plsc.subcore_barrier()

    def fire(j, carry):
        pltpu.async_copy(ones_v, acc_sh.at[idx_v.at[j]], sem, add=True)
        return carry

    lax.fori_loop(0, k, fire, 0)

    def drain(j, carry):
        pltpu.make_async_copy(ones_v, acc_sh.at[idx_v.at[j]], sem).wait()
        return carry

    lax.fori_loop(0, k, drain, 0)
    plsc.subcore_barrier()
    pltpu.sync_copy(
        acc_sh.at[pl.ds(s * stripe, stripe)],
        out_hbm.at[c, pl.ds(s * stripe, stripe)],
    )


def _make_degree(n_pad, k):
    return pl.kernel(
        _degree_body,
        out_type=jax.ShapeDtypeStruct((NC, n_pad), jnp.float32),
        mesh=_sc_mesh(),
        scratch_types=[
            pltpu.VMEM((k, EB), jnp.int32),
            pltpu.VMEM((EB,), jnp.float32),
            pltpu.VMEM_SHARED((n_pad,), jnp.float32),
            pltpu.SemaphoreType.DMA,
        ],
    )


def _scatter_body(table_hbm, src_hbm, dst_hbm, zeros_hbm, out_hbm,
                  sidx_v, didx_v, rows_a, rows_b, acc_sh,
                  gsem_a, gsem_b, ssem_a, ssem_b):
    c = lax.axis_index("c")
    s = lax.axis_index("s")
    wid = c * NS + s
    kh = sidx_v.shape[0]
    stripe = acc_sh.shape[0] // NS

    for h in range(2):
        pltpu.sync_copy(src_hbm.at[wid, pl.ds(h * kh, kh)], sidx_v)
        pltpu.sync_copy(dst_hbm.at[wid, pl.ds(h * kh, kh)], didx_v)
        pltpu.async_copy(table_hbm.at[sidx_v.at[0]], rows_a, gsem_a)
        pltpu.async_copy(table_hbm.at[sidx_v.at[1]], rows_b, gsem_b)
        if h == 0:
            pltpu.sync_copy(zeros_hbm, acc_sh.at[pl.ds(s * stripe, stripe)])
            plsc.subcore_barrier()

        def body(jj, carry):
            j0 = 2 * jj
            pltpu.make_async_copy(table_hbm.at[sidx_v.at[j0]], rows_a, gsem_a).wait()
            pltpu.async_copy(rows_a, acc_sh.at[didx_v.at[j0]], ssem_a, add=True)
            pltpu.make_async_copy(table_hbm.at[sidx_v.at[j0 + 1]], rows_b, gsem_b).wait()
            pltpu.async_copy(rows_b, acc_sh.at[didx_v.at[j0 + 1]], ssem_b, add=True)

            @pl.when(jj + 1 < kh // 2)
            def _():
                pltpu.make_async_copy(rows_a, acc_sh.at[didx_v.at[j0]], ssem_a).wait()
                pltpu.async_copy(table_hbm.at[sidx_v.at[j0 + 2]], rows_a, gsem_a)
                pltpu.make_async_copy(rows_b, acc_sh.at[didx_v.at[j0 + 1]], ssem_b).wait()
                pltpu.async_copy(table_hbm.at[sidx_v.at[j0 + 3]], rows_b, gsem_b)

            return carry

        lax.fori_loop(0, kh // 2, body, 0)
        pltpu.make_async_copy(rows_a, acc_sh.at[didx_v.at[kh - 2]], ssem_a).wait()
        pltpu.make_async_copy(rows_b, acc_sh.at[didx_v.at[kh - 1]], ssem_b).wait()

    plsc.subcore_barrier()
    pltpu.sync_copy(
        acc_sh.at[pl.ds(s * stripe, stripe)],
        out_hbm.at[c, pl.ds(s * stripe, stripe)],
    )


def _make_scatter(n_pad, feat, k):
    return pl.kernel(
        _scatter_body,
        out_type=jax.ShapeDtypeStruct((NC, n_pad, feat), jnp.float32),
        mesh=_sc_mesh(),
        scratch_types=[
            pltpu.VMEM((k // 2, EB), jnp.int32),
            pltpu.VMEM((k // 2, EB), jnp.int32),
            pltpu.VMEM((EB, feat), jnp.float32),
            pltpu.VMEM((EB, feat), jnp.float32),
            pltpu.VMEM_SHARED((n_pad, feat), jnp.float32),
            pltpu.SemaphoreType.DMA,
            pltpu.SemaphoreType.DMA,
            pltpu.SemaphoreType.DMA,
            pltpu.SemaphoreType.DMA,
        ],
        compiler_params=pltpu.CompilerParams(use_tc_tiling_on_sc=False),
    )


def _prologue_body(deg0_ref, deg1_ref, x_ref, w_ref, dis_ref, h_ref):
    deg = deg0_ref[...] + deg1_ref[...] + 1.0
    dis = lax.rsqrt(deg)
    dis_ref[...] = dis
    h_ref[...] = jnp.dot(
        x_ref[...], w_ref[...], preferred_element_type=jnp.float32
    ) * dis


def _mid_body(s0_ref, s1_ref, h_ref, dis_ref, b_ref, w_ref, o_ref, *, blk, n_valid):
    i = pl.program_id(0)
    dis = dis_ref[...]
    z = dis * (s0_ref[...] + s1_ref[...] + h_ref[...]) + b_ref[...]
    z = jnp.maximum(z, 0.0)
    rows = i * blk + lax.broadcasted_iota(jnp.int32, (blk, 1), 0)
    z = jnp.where(rows < n_valid, z, 0.0)
    o_ref[...] = jnp.dot(
        z, w_ref[...], preferred_element_type=jnp.float32
    ) * dis


def _epilogue_body(s0_ref, s1_ref, h_ref, dis_ref, b_ref, o_ref):
    o_ref[...] = dis_ref[...] * (s0_ref[...] + s1_ref[...] + h_ref[...]) + b_ref[...]


def _row_spec(blk, width):
    return pl.BlockSpec((blk, width), lambda i: (i, 0))


def _const_spec(shape):
    return pl.BlockSpec(shape, lambda i: (0, 0))


def _run_prologue(deg0, deg1, x_pad, w, blk):
    n_pad, fin = x_pad.shape
    h = w.shape[1]
    grid = n_pad // blk
    return pl.pallas_call(
        _prologue_body,
        grid=(grid,),
        in_specs=[
            _row_spec(blk, 1),
            _row_spec(blk, 1),
            _row_spec(blk, fin),
            _const_spec((fin, h)),
        ],
        out_specs=[_row_spec(blk, 1), _row_spec(blk, h)],
        out_shape=[
            jax.ShapeDtypeStruct((n_pad, 1), jnp.float32),
            jax.ShapeDtypeStruct((n_pad, h), jnp.float32),
        ],
    )(deg0, deg1, x_pad, w)


def _run_mid(s_parts, h1p, dis, b1, w2p, blk, n_valid):
    n_pad, h = h1p.shape
    cp = w2p.shape[1]
    grid = n_pad // blk
    return pl.pallas_call(
        functools.partial(_mid_body, blk=blk, n_valid=n_valid),
        grid=(grid,),
        in_specs=[
            _row_spec(blk, h),
            _row_spec(blk, h),
            _row_spec(blk, h),
            _row_spec(blk, 1),
            _const_spec((1, h)),
            _const_spec((h, cp)),
        ],
        out_specs=_row_spec(blk, cp),
        out_shape=jax.ShapeDtypeStruct((n_pad, cp), jnp.float32),
    )(s_parts[0], s_parts[1], h1p, dis, b1, w2p)


def _run_epilogue(s_parts, h2p, dis, b2p, blk):
    n_pad, cp = h2p.shape
    grid = n_pad // blk
    return pl.pallas_call(
        _epilogue_body,
        grid=(grid,),
        in_specs=[
            _row_spec(blk, cp),
            _row_spec(blk, cp),
            _row_spec(blk, cp),
            _row_spec(blk, 1),
            _const_spec((1, cp)),
        ],
        out_specs=_row_spec(blk, cp),
        out_shape=jax.ShapeDtypeStruct((n_pad, cp), jnp.float32),
    )(s_parts[0], s_parts[1], h2p, dis, b2p)


def _round_up(a, b):
    return (a + b - 1) // b * b


def kernel(x, edge_index, W1, b1, W2, b2):
    n, fin = x.shape
    hid = W1.shape[1]
    ncls = W2.shape[1]
    e = edge_index.shape[1]

    blk = 256
    n_pad = _round_up(n + 1, max(blk, NS * 8))
    e_pad = _round_up(e, NW * EB * 4)
    k = e_pad // (NW * EB)
    cp = _round_up(ncls, 16)

    src = edge_index[0].astype(jnp.int32)
    dst = edge_index[1].astype(jnp.int32)
    pad_idx = jnp.full((e_pad - e,), n, dtype=jnp.int32)
    src3 = jnp.concatenate([src, pad_idx]).reshape(NW, k, EB)
    dst3 = jnp.concatenate([dst, pad_idx]).reshape(NW, k, EB)

    x_pad = jnp.zeros((n_pad, fin), jnp.float32).at[:n].set(x)
    w2p = jnp.zeros((hid, cp), jnp.float32).at[:, :ncls].set(W2)
    b1r = b1.reshape(1, hid)
    b2p = jnp.zeros((1, cp), jnp.float32).at[0, :ncls].set(b2)

    stripe = n_pad // NS
    zeros1 = jnp.zeros((stripe,), jnp.float32)
    ones_eb = jnp.ones((EB,), jnp.float32)
    zeros_h = jnp.zeros((stripe, hid), jnp.float32)
    zeros_c = jnp.zeros((stripe, cp), jnp.float32)

    degp = _make_degree(n_pad, k)(dst3, ones_eb, zeros1)
    deg0 = degp[0].reshape(n_pad, 1)
    deg1 = degp[1].reshape(n_pad, 1)

    dis, h1p = _run_prologue(deg0, deg1, x_pad, W1, blk)
    s1 = _make_scatter(n_pad, hid, k)(h1p, src3, dst3, zeros_h)
    h2p = _run_mid(s1, h1p, dis, b1r, w2p, blk, n)
    s2 = _make_scatter(n_pad, cp, k)(h2p, src3, dst3, zeros_c)
    out = _run_epilogue(s2, h2p, dis, b2p, blk)
    return out[:n, :ncls]


# speedup vs baseline: 2.3751x; 2.3751x over previous
import functools

import jax
import jax.numpy as jnp
from jax import lax
from jax.experimental import pallas as pl
from jax.experimental.pallas import tpu as pltpu
from jax.experimental.pallas import tpu_sc as plsc

NC = 2
NS = 16
NW = NC * NS
EB = 128


def _sc_mesh():
    return plsc.VectorSubcoreMesh(
        core_axis_name="c", subcore_axis_name="s", num_cores=NC, num_subcores=NS
    )


def _degree_body(dst_hbm, ones_hbm, zeros_hbm, out_hbm, idx_v, ones_v, acc_sh, sem):
    c = lax.axis_index("c")
    s = lax.axis_index("s")
    wid = c * NS + s
    k = idx_v.shape[0]
    stripe = acc_sh.shape[0] // NS

    pltpu.sync_copy(dst_hbm.at[wid], idx_v)
    pltpu.sync_copy(ones_hbm, ones_v)
    pltpu.sync_copy(zeros_hbm, acc_sh.at[pl.ds(s * stripe, stripe)])
    plsc.subcore_barrier()

    def fire(j, carry):
        pltpu.async_copy(ones_v, acc_sh.at[idx_v.at[j]], sem, add=True)
        return carry

    lax.fori_loop(0, k, fire, 0)

    def drain(j, carry):
        pltpu.make_async_copy(ones_v, acc_sh.at[idx_v.at[j]], sem).wait()
        return carry

    lax.fori_loop(0, k, drain, 0)
    plsc.subcore_barrier()
    pltpu.sync_copy(
        acc_sh.at[pl.ds(s * stripe, stripe)],
        out_hbm.at[c, pl.ds(s * stripe, stripe)],
    )


def _make_degree(n_pad, k):
    return pl.kernel(
        _degree_body,
        out_type=jax.ShapeDtypeStruct((NC, n_pad), jnp.float32),
        mesh=_sc_mesh(),
        scratch_types=[
            pltpu.VMEM((k, EB), jnp.int32),
            pltpu.VMEM((EB,), jnp.float32),
            pltpu.VMEM_SHARED((n_pad,), jnp.float32),
            pltpu.SemaphoreType.DMA,
        ],
    )


def _scatter_body(table_hbm, src_hbm, dst_hbm, zeros_hbm, out_hbm,
                  sidx_v, didx_v, rows_a, rows_b, acc_sh,
                  gsem_a, gsem_b, ssem_a, ssem_b):
    c = lax.axis_index("c")
    s = lax.axis_index("s")
    wid = c * NS + s
    kh = sidx_v.shape[0]
    stripe = acc_sh.shape[0] // NS

    for h in range(2):
        pltpu.sync_copy(src_hbm.at[wid, pl.ds(h * kh, kh)], sidx_v)
        pltpu.sync_copy(dst_hbm.at[wid, pl.ds(h * kh, kh)], didx_v)
        pltpu.async_copy(table_hbm.at[sidx_v.at[0]], rows_a, gsem_a)
        pltpu.async_copy(table_hbm.at[sidx_v.at[1]], rows_b, gsem_b)
        if h == 0:
            pltpu.sync_copy(zeros_hbm, acc_sh.at[pl.ds(s * stripe, stripe)])
            plsc.subcore_barrier()

        def body(jj, carry):
            j0 = 2 * jj
            pltpu.make_async_copy(table_hbm.at[sidx_v.at[j0]], rows_a, gsem_a).wait()
            pltpu.async_copy(rows_a, acc_sh.at[didx_v.at[j0]], ssem_a, add=True)
            pltpu.make_async_copy(table_hbm.at[sidx_v.at[j0 + 1]], rows_b, gsem_b).wait()
            pltpu.async_copy(rows_b, acc_sh.at[didx_v.at[j0 + 1]], ssem_b, add=True)

            @pl.when(jj + 1 < kh // 2)
            def _():
                pltpu.make_async_copy(rows_a, acc_sh.at[didx_v.at[j0]], ssem_a).wait()
                pltpu.async_copy(table_hbm.at[sidx_v.at[j0 + 2]], rows_a, gsem_a)
                pltpu.make_async_copy(rows_b, acc_sh.at[didx_v.at[j0 + 1]], ssem_b).wait()
                pltpu.async_copy(table_hbm.at[sidx_v.at[j0 + 3]], rows_b, gsem_b)

            return carry

        lax.fori_loop(0, kh // 2, body, 0)
        pltpu.make_async_copy(rows_a, acc_sh.at[didx_v.at[kh - 2]], ssem_a).wait()
        pltpu.make_async_copy(rows_b, acc_sh.at[didx_v.at[kh - 1]], ssem_b).wait()

    plsc.subcore_barrier()
    pltpu.sync_copy(
        acc_sh.at[pl.ds(s * stripe, stripe)],
        out_hbm.at[c, pl.ds(s * stripe, stripe)],
    )


def _make_scatter(n_pad, feat, k):
    return pl.kernel(
        _scatter_body,
        out_type=jax.ShapeDtypeStruct((NC, n_pad, feat), jnp.float32),
        mesh=_sc_mesh(),
        scratch_types=[
            pltpu.VMEM((k // 2, EB), jnp.int32),
            pltpu.VMEM((k // 2, EB), jnp.int32),
            pltpu.VMEM((EB, feat), jnp.float32),
            pltpu.VMEM((EB, feat), jnp.float32),
            pltpu.VMEM_SHARED((n_pad, feat), jnp.float32),
            pltpu.SemaphoreType.DMA,
            pltpu.SemaphoreType.DMA,
            pltpu.SemaphoreType.DMA,
            pltpu.SemaphoreType.DMA,
        ],
        compiler_params=pltpu.CompilerParams(use_tc_tiling_on_sc=False),
    )


def _prologue_body(deg0_ref, deg1_ref, x_ref, w_ref, dis_ref, h_ref):
    deg = deg0_ref[...] + deg1_ref[...] + 1.0
    dis = lax.rsqrt(deg)
    dis_ref[...] = dis
    h_ref[...] = jnp.dot(
        x_ref[...], w_ref[...], preferred_element_type=jnp.float32
    ) * dis


def _mid_body(s0_ref, s1_ref, h_ref, dis_ref, b_ref, w_ref, o_ref, *, blk, n_valid):
    i = pl.program_id(0)
    dis = dis_ref[...]
    z = dis * (s0_ref[...] + s1_ref[...] + h_ref[...]) + b_ref[...]
    z = jnp.maximum(z, 0.0)
    rows = i * blk + lax.broadcasted_iota(jnp.int32, (blk, 1), 0)
    z = jnp.where(rows < n_valid, z, 0.0)
    o_ref[...] = jnp.dot(
        z, w_ref[...], preferred_element_type=jnp.float32
    ) * dis


def _epilogue_body(s0_ref, s1_ref, h_ref, dis_ref, b_ref, o_ref):
    o_ref[...] = dis_ref[...] * (s0_ref[...] + s1_ref[...] + h_ref[...]) + b_ref[...]


def _row_spec(blk, width):
    return pl.BlockSpec((blk, width), lambda i: (i, 0))


def _const_spec(shape):
    return pl.BlockSpec(shape, lambda i: (0, 0))


def _run_prologue(deg0, deg1, x_pad, w, blk):
    n_pad, fin = x_pad.shape
    h = w.shape[1]
    grid = n_pad // blk
    return pl.pallas_call(
        _prologue_body,
        grid=(grid,),
        in_specs=[
            _row_spec(blk, 1),
            _row_spec(blk, 1),
            _row_spec(blk, fin),
            _const_spec((fin, h)),
        ],
        out_specs=[_row_spec(blk, 1), _row_spec(blk, h)],
        out_shape=[
            jax.ShapeDtypeStruct((n_pad, 1), jnp.float32),
            jax.ShapeDtypeStruct((n_pad, h), jnp.float32),
        ],
    )(deg0, deg1, x_pad, w)


def _run_mid(s_parts, h1p, dis, b1, w2p, blk, n_valid):
    n_pad, h = h1p.shape
    cp = w2p.shape[1]
    grid = n_pad // blk
    return pl.pallas_call(
        functools.partial(_mid_body, blk=blk, n_valid=n_valid),
        grid=(grid,),
        in_specs=[
            _row_spec(blk, h),
            _row_spec(blk, h),
            _row_spec(blk, h),
            _row_spec(blk, 1),
            _const_spec((1, h)),
            _const_spec((h, cp)),
        ],
        out_specs=_row_spec(blk, cp),
        out_shape=jax.ShapeDtypeStruct((n_pad, cp), jnp.float32),
    )(s_parts[0], s_parts[1], h1p, dis, b1, w2p)


def _run_epilogue(s_parts, h2p, dis, b2p, blk):
    n_pad, cp = h2p.shape
    grid = n_pad // blk
    return pl.pallas_call(
        _epilogue_body,
        grid=(grid,),
        in_specs=[
            _row_spec(blk, cp),
            _row_spec(blk, cp),
            _row_spec(blk, cp),
            _row_spec(blk, 1),
            _const_spec((1, cp)),
        ],
        out_specs=_row_spec(blk, cp),
        out_shape=jax.ShapeDtypeStruct((n_pad, cp), jnp.float32),
    )(s_parts[0], s_parts[1], h2p, dis, b2p)


def _round_up(a, b):
    return (a + b - 1) // b * b


def kernel(x, edge_index, W1, b1, W2, b2):
    n, fin = x.shape
    hid = W1.shape[1]
    ncls = W2.shape[1]
    e = edge_index.shape[1]

    blk = 256
    n_pad = _round_up(n + 1, max(blk, NS * 8))
    e_pad = _round_up(e, NW * EB * 4)
    k = e_pad // (NW * EB)
    cp = _round_up(ncls, 16)

    src = edge_index[0].astype(jnp.int32)
    dst = edge_index[1].astype(jnp.int32)
    pad_idx = n + jnp.arange(e_pad - e, dtype=jnp.int32) % (n_pad - n)
    src3 = jnp.concatenate([src, pad_idx]).reshape(NW, k, EB)
    dst3 = jnp.concatenate([dst, pad_idx]).reshape(NW, k, EB)

    x_pad = jnp.zeros((n_pad, fin), jnp.float32).at[:n].set(x)
    w2p = jnp.zeros((hid, cp), jnp.float32).at[:, :ncls].set(W2)
    b1r = b1.reshape(1, hid)
    b2p = jnp.zeros((1, cp), jnp.float32).at[0, :ncls].set(b2)

    stripe = n_pad // NS
    zeros1 = jnp.zeros((stripe,), jnp.float32)
    ones_eb = jnp.ones((EB,), jnp.float32)
    zeros_h = jnp.zeros((stripe, hid), jnp.float32)
    zeros_c = jnp.zeros((stripe, cp), jnp.float32)

    degp = _make_degree(n_pad, k)(dst3, ones_eb, zeros1)
    deg0 = degp[0].reshape(n_pad, 1)
    deg1 = degp[1].reshape(n_pad, 1)

    dis, h1p = _run_prologue(deg0, deg1, x_pad, W1, blk)
    s1 = _make_scatter(n_pad, hid, k)(h1p, src3, dst3, zeros_h)
    h2p = _run_mid(s1, h1p, dis, b1r, w2p, blk, n)
    s2 = _make_scatter(n_pad, cp, k)(h2p, src3, dst3, zeros_c)
    out = _run_epilogue(s2, h2p, dis, b2p, blk)
    return out[:n, :ncls]


# speedup vs baseline: 3.1596x; 1.3303x over previous
import functools

import jax
import jax.numpy as jnp
from jax import lax
from jax.experimental import pallas as pl
from jax.experimental.pallas import tpu as pltpu
from jax.experimental.pallas import tpu_sc as plsc

NC = 2
NS = 16
NW = NC * NS
EB = 128


def _sc_mesh():
    return plsc.VectorSubcoreMesh(
        core_axis_name="c", subcore_axis_name="s", num_cores=NC, num_subcores=NS
    )


def _degree_body(dst_hbm, ones_hbm, zeros_hbm, out_hbm, idx_v, ones_v, acc_sh, sem):
    c = lax.axis_index("c")
    s = lax.axis_index("s")
    wid = c * NS + s
    k = idx_v.shape[0]
    stripe = acc_sh.shape[0] // NS

    pltpu.sync_copy(dst_hbm.at[wid], idx_v)
    pltpu.sync_copy(ones_hbm, ones_v)
    pltpu.sync_copy(zeros_hbm, acc_sh.at[pl.ds(s * stripe, stripe)])
    plsc.subcore_barrier()

    def fire(j, carry):
        pltpu.async_copy(ones_v, acc_sh.at[idx_v.at[j]], sem, add=True)
        return carry

    lax.fori_loop(0, k, fire, 0)

    def drain(j, carry):
        pltpu.make_async_copy(ones_v, acc_sh.at[idx_v.at[j]], sem).wait()
        return carry

    lax.fori_loop(0, k, drain, 0)
    plsc.subcore_barrier()
    pltpu.sync_copy(
        acc_sh.at[pl.ds(s * stripe, stripe)],
        out_hbm.at[c, pl.ds(s * stripe, stripe)],
    )


def _make_degree(n_pad, k):
    return pl.kernel(
        _degree_body,
        out_type=jax.ShapeDtypeStruct((NC, n_pad), jnp.float32),
        mesh=_sc_mesh(),
        scratch_types=[
            pltpu.VMEM((k, EB), jnp.int32),
            pltpu.VMEM((EB,), jnp.float32),
            pltpu.VMEM_SHARED((n_pad,), jnp.float32),
            pltpu.SemaphoreType.DMA,
        ],
    )


def _scatter_body(table_hbm, src_hbm, dst_hbm, zeros_hbm, out_hbm,
                  sidx_v, didx_v, rows_a, rows_b, acc_sh,
                  gsem_a, gsem_b, ssem_a, ssem_b):
    c = lax.axis_index("c")
    s = lax.axis_index("s")
    wid = c * NS + s
    kh = sidx_v.shape[0]
    stripe = acc_sh.shape[0] // NS

    for h in range(2):
        pltpu.sync_copy(src_hbm.at[wid, pl.ds(h * kh, kh)], sidx_v)
        pltpu.sync_copy(dst_hbm.at[wid, pl.ds(h * kh, kh)], didx_v)
        pltpu.async_copy(table_hbm.at[sidx_v.at[0]], rows_a, gsem_a)
        pltpu.async_copy(table_hbm.at[sidx_v.at[1]], rows_b, gsem_b)
        if h == 0:
            pltpu.sync_copy(zeros_hbm, acc_sh.at[pl.ds(s * stripe, stripe)])
            plsc.subcore_barrier()

        def body(jj, carry):
            j0 = 2 * jj
            pltpu.make_async_copy(table_hbm.at[sidx_v.at[j0]], rows_a, gsem_a).wait()
            pltpu.async_copy(rows_a, acc_sh.at[didx_v.at[j0]], ssem_a, add=True)
            pltpu.make_async_copy(table_hbm.at[sidx_v.at[j0 + 1]], rows_b, gsem_b).wait()
            pltpu.async_copy(rows_b, acc_sh.at[didx_v.at[j0 + 1]], ssem_b, add=True)

            @pl.when(jj + 1 < kh // 2)
            def _():
                pltpu.make_async_copy(rows_a, acc_sh.at[didx_v.at[j0]], ssem_a).wait()
                pltpu.async_copy(table_hbm.at[sidx_v.at[j0 + 2]], rows_a, gsem_a)
                pltpu.make_async_copy(rows_b, acc_sh.at[didx_v.at[j0 + 1]], ssem_b).wait()
                pltpu.async_copy(table_hbm.at[sidx_v.at[j0 + 3]], rows_b, gsem_b)

            return carry

        lax.fori_loop(0, kh // 2, body, 0)
        pltpu.make_async_copy(rows_a, acc_sh.at[didx_v.at[kh - 2]], ssem_a).wait()
        pltpu.make_async_copy(rows_b, acc_sh.at[didx_v.at[kh - 1]], ssem_b).wait()

    plsc.subcore_barrier()
    pltpu.sync_copy(
        acc_sh.at[pl.ds(s * stripe, stripe)],
        out_hbm.at[c, pl.ds(s * stripe, stripe)],
    )


def _make_scatter(n_pad, feat, k):
    return pl.kernel(
        _scatter_body,
        out_type=jax.ShapeDtypeStruct((NC, n_pad, feat), jnp.float32),
        mesh=_sc_mesh(),
        scratch_types=[
            pltpu.VMEM((k // 2, EB), jnp.int32),
            pltpu.VMEM((k // 2, EB), jnp.int32),
            pltpu.VMEM((EB, feat), jnp.float32),
            pltpu.VMEM((EB, feat), jnp.float32),
            pltpu.VMEM_SHARED((n_pad, feat), jnp.float32),
            pltpu.SemaphoreType.DMA,
            pltpu.SemaphoreType.DMA,
            pltpu.SemaphoreType.DMA,
            pltpu.SemaphoreType.DMA,
        ],
        compiler_params=pltpu.CompilerParams(use_tc_tiling_on_sc=False),
    )


def _prologue_body(deg0_ref, deg1_ref, x_ref, w_ref, dis_ref, h_ref):
    deg = deg0_ref[...] + deg1_ref[...] + 1.0
    dis = lax.rsqrt(deg)
    dis_ref[...] = dis
    h_ref[...] = jnp.dot(
        x_ref[...], w_ref[...], preferred_element_type=jnp.float32
    ) * dis


def _mid_body(s0_ref, s1_ref, h_ref, dis_ref, b_ref, w_ref, o_ref, *, blk, n_valid):
    i = pl.program_id(0)
    dis = dis_ref[...]
    z = dis * (s0_ref[0] + s1_ref[0] + h_ref[...]) + b_ref[...]
    z = jnp.maximum(z, 0.0)
    rows = i * blk + lax.broadcasted_iota(jnp.int32, (blk, 1), 0)
    z = jnp.where(rows < n_valid, z, 0.0)
    o_ref[...] = jnp.dot(
        z, w_ref[...], preferred_element_type=jnp.float32
    ) * dis


def _epilogue_body(s0_ref, s1_ref, h_ref, dis_ref, b_ref, o_ref):
    o_ref[...] = dis_ref[...] * (s0_ref[0] + s1_ref[0] + h_ref[...]) + b_ref[...]


def _row_spec(blk, width):
    return pl.BlockSpec((blk, width), lambda i: (i, 0))


def _const_spec(shape):
    return pl.BlockSpec(shape, lambda i: (0, 0))


def _run_prologue(deg0, deg1, x_pad, w, blk):
    n_pad, fin = x_pad.shape
    h = w.shape[1]
    grid = n_pad // blk
    return pl.pallas_call(
        _prologue_body,
        grid=(grid,),
        in_specs=[
            _row_spec(blk, 1),
            _row_spec(blk, 1),
            _row_spec(blk, fin),
            _const_spec((fin, h)),
        ],
        out_specs=[_row_spec(blk, 1), _row_spec(blk, h)],
        out_shape=[
            jax.ShapeDtypeStruct((n_pad, 1), jnp.float32),
            jax.ShapeDtypeStruct((n_pad, h), jnp.float32),
        ],
    )(deg0, deg1, x_pad, w)


def _part_spec(core, blk, width):
    return pl.BlockSpec((1, blk, width), lambda i, c=core: (c, i, 0))


def _run_mid(s_parts, h1p, dis, b1, w2p, blk, n_valid):
    n_pad, h = h1p.shape
    cp = w2p.shape[1]
    grid = n_pad // blk
    return pl.pallas_call(
        functools.partial(_mid_body, blk=blk, n_valid=n_valid),
        grid=(grid,),
        in_specs=[
            _part_spec(0, blk, h),
            _part_spec(1, blk, h),
            _row_spec(blk, h),
            _row_spec(blk, 1),
            _const_spec((1, h)),
            _const_spec((h, cp)),
        ],
        out_specs=_row_spec(blk, cp),
        out_shape=jax.ShapeDtypeStruct((n_pad, cp), jnp.float32),
    )(s_parts, s_parts, h1p, dis, b1, w2p)


def _run_epilogue(s_parts, h2p, dis, b2p, blk):
    n_pad, cp = h2p.shape
    grid = n_pad // blk
    return pl.pallas_call(
        _epilogue_body,
        grid=(grid,),
        in_specs=[
            _part_spec(0, blk, cp),
            _part_spec(1, blk, cp),
            _row_spec(blk, cp),
            _row_spec(blk, 1),
            _const_spec((1, cp)),
        ],
        out_specs=_row_spec(blk, cp),
        out_shape=jax.ShapeDtypeStruct((n_pad, cp), jnp.float32),
    )(s_parts, s_parts, h2p, dis, b2p)


def _round_up(a, b):
    return (a + b - 1) // b * b


def kernel(x, edge_index, W1, b1, W2, b2):
    n, fin = x.shape
    hid = W1.shape[1]
    ncls = W2.shape[1]
    e = edge_index.shape[1]

    blk = 1024
    n_pad = _round_up(n + 1, max(blk, NS * 8))
    e_pad = _round_up(e, NW * EB * 4)
    k = e_pad // (NW * EB)
    cp = _round_up(ncls, 16)

    src = edge_index[0].astype(jnp.int32)
    dst = edge_index[1].astype(jnp.int32)
    pad_idx = n + jnp.arange(e_pad - e, dtype=jnp.int32) % (n_pad - n)
    src3 = jnp.concatenate([src, pad_idx]).reshape(NW, k, EB)
    dst3 = jnp.concatenate([dst, pad_idx]).reshape(NW, k, EB)

    x_pad = jnp.zeros((n_pad, fin), jnp.float32).at[:n].set(x)
    w2p = jnp.zeros((hid, cp), jnp.float32).at[:, :ncls].set(W2)
    b1r = b1.reshape(1, hid)
    b2p = jnp.zeros((1, cp), jnp.float32).at[0, :ncls].set(b2)

    stripe = n_pad // NS
    zeros1 = jnp.zeros((stripe,), jnp.float32)
    ones_eb = jnp.ones((EB,), jnp.float32)
    zeros_h = jnp.zeros((stripe, hid), jnp.float32)
    zeros_c = jnp.zeros((stripe, cp), jnp.float32)

    degp = _make_degree(n_pad, k)(dst3, ones_eb, zeros1)
    deg0 = degp[0].reshape(n_pad, 1)
    deg1 = degp[1].reshape(n_pad, 1)

    dis, h1p = _run_prologue(deg0, deg1, x_pad, W1, blk)
    s1 = _make_scatter(n_pad, hid, k)(h1p, src3, dst3, zeros_h)
    h2p = _run_mid(s1, h1p, dis, b1r, w2p, blk, n)
    s2 = _make_scatter(n_pad, cp, k)(h2p, src3, dst3, zeros_c)
    out = _run_epilogue(s2, h2p, dis, b2p, blk)
    return out[:n, :ncls]
